# Initial kernel scaffold; baseline (speedup 1.0000x reference)
import jax
import jax.numpy as jnp
from jax.experimental import pallas as pl


def kernel(x, embed_weight, pe):
    raise NotImplementedError("write your pallas kernel here")


# speedup vs baseline: 2.7250x; 2.7250x over previous
import functools

import jax
import jax.numpy as jnp
from jax import lax
from jax.experimental import pallas as pl
from jax.experimental.pallas import tpu as pltpu
from jax.experimental.pallas import tpu_sc as plsc

MAX_LEN = 200
EMBED_DIM = 128
BATCH = 1024
NUM_CORES = 2
NUM_SUBCORES = 16
NUM_WORKERS = NUM_CORES * NUM_SUBCORES
CHUNK = 40
IDX_PER_WORKER = BATCH * MAX_LEN // NUM_WORKERS
NUM_CHUNKS = IDX_PER_WORKER // CHUNK
PE_PHASES = MAX_LEN // CHUNK
UNROLL = 2 * PE_PHASES
LANES = 16


def _sc_embed(idx, table, pe2d):
    mesh = plsc.VectorSubcoreMesh(core_axis_name="c", subcore_axis_name="s")

    @functools.partial(
        pl.kernel,
        mesh=mesh,
        out_type=jax.ShapeDtypeStruct((BATCH * MAX_LEN, EMBED_DIM), jnp.float32),
        scratch_types=[
            pltpu.VMEM((NUM_CHUNKS, CHUNK), jnp.int32),
            pltpu.VMEM((CHUNK, EMBED_DIM), jnp.float32),
            pltpu.VMEM((CHUNK, EMBED_DIM), jnp.float32),
            pltpu.SemaphoreType.DMA,
            pltpu.SemaphoreType.DMA,
        ],
    )
    def k(idx_hbm, table_hbm, pe_hbm, out_hbm, idx_v, buf0, buf1, sem0, sem1):
        wid = lax.axis_index("s") * NUM_CORES + lax.axis_index("c")
        base = wid * IDX_PER_WORKER
        pltpu.sync_copy(idx_hbm.at[wid], idx_v)

        bufs = (buf0, buf1)
        sems = (sem0, sem1)

        pltpu.sync_copy(pe_hbm.at[pl.ds(0, CHUNK)], buf0)
        pltpu.async_copy(table_hbm.at[idx_v.at[0]], buf0, sem0, add=True)

        @pl.loop(0, NUM_CHUNKS, step=UNROLL)
        def _(g):
            for b in range(UNROLL):
                gg = g + b
                buf, sem = bufs[b % 2], sems[b % 2]
                nbuf, nsem = bufs[1 - b % 2], sems[1 - b % 2]
                np0 = CHUNK * ((b + 1) % PE_PHASES)

                @pl.when(gg + 1 < NUM_CHUNKS)
                def _():
                    pltpu.sync_copy(pe_hbm.at[pl.ds(np0, CHUNK)], nbuf)
                    pltpu.async_copy(table_hbm.at[idx_v.at[gg + 1]], nbuf, nsem,
                                     add=True)

                pltpu.make_async_copy(table_hbm.at[idx_v.at[gg]], buf, sem).wait()
                pltpu.sync_copy(buf, out_hbm.at[pl.ds(base + gg * CHUNK, CHUNK)])

    return k(idx, table, pe2d)


def kernel(x, embed_weight, pe):
    idx = x.astype(jnp.int32).reshape(NUM_WORKERS, NUM_CHUNKS, CHUNK)
    pe2d = pe.reshape(MAX_LEN, EMBED_DIM)
    out = _sc_embed(idx, embed_weight, pe2d)
    return out.reshape(BATCH, MAX_LEN, EMBED_DIM)


# speedup vs baseline: 4.3558x; 1.5985x over previous
import functools

import jax
import jax.numpy as jnp
from jax import lax
from jax.experimental import pallas as pl
from jax.experimental.pallas import tpu as pltpu
from jax.experimental.pallas import tpu_sc as plsc

MAX_LEN = 200
EMBED_DIM = 128
BATCH = 1024
NUM_CORES = 2
NUM_SUBCORES = 16
NUM_WORKERS = NUM_CORES * NUM_SUBCORES
CHUNK = 128
IDX_PER_WORKER = BATCH * MAX_LEN // NUM_WORKERS
NUM_CHUNKS = IDX_PER_WORKER // CHUNK
UNROLL = 2
LANES = 16


def _sc_embed(idx, table, pe2d):
    mesh = plsc.VectorSubcoreMesh(core_axis_name="c", subcore_axis_name="s")

    @functools.partial(
        pl.kernel,
        mesh=mesh,
        out_type=jax.ShapeDtypeStruct((BATCH * MAX_LEN, EMBED_DIM), jnp.float32),
        scratch_types=[
            pltpu.VMEM((NUM_CHUNKS, CHUNK), jnp.int32),
            pltpu.VMEM((CHUNK, EMBED_DIM), jnp.float32),
            pltpu.VMEM((CHUNK, EMBED_DIM), jnp.float32),
            pltpu.SemaphoreType.DMA,
            pltpu.SemaphoreType.DMA,
        ],
    )
    def k(idx_hbm, table_hbm, pe_hbm, out_hbm, idx_v, buf0, buf1, sem0, sem1):
        wid = lax.axis_index("s") * NUM_CORES + lax.axis_index("c")
        base = wid * IDX_PER_WORKER
        pltpu.sync_copy(idx_hbm.at[wid], idx_v)

        bufs = (buf0, buf1)
        sems = (sem0, sem1)

        pltpu.sync_copy(pe_hbm.at[pl.ds(0, CHUNK)], buf0)
        pltpu.async_copy(table_hbm.at[idx_v.at[0]], buf0, sem0, add=True)

        @pl.loop(0, NUM_CHUNKS, step=UNROLL)
        def _(g):
            for b in range(UNROLL):
                gg = g + b
                buf, sem = bufs[b % 2], sems[b % 2]
                nbuf, nsem = bufs[1 - b % 2], sems[1 - b % 2]

                @pl.when(gg + 1 < NUM_CHUNKS)
                def _():
                    pltpu.sync_copy(pe_hbm.at[pl.ds((gg + 1) * CHUNK, CHUNK)],
                                    nbuf)
                    pltpu.async_copy(table_hbm.at[idx_v.at[gg + 1]], nbuf, nsem,
                                     add=True)

                pltpu.make_async_copy(table_hbm.at[idx_v.at[gg]], buf, sem).wait()
                pltpu.sync_copy(buf, out_hbm.at[pl.ds(base + gg * CHUNK, CHUNK)])

    return k(idx, table, pe2d)


def kernel(x, embed_weight, pe):
    idx = x.astype(jnp.int32).reshape(NUM_WORKERS, NUM_CHUNKS, CHUNK)
    pe_rep = jnp.tile(pe.reshape(MAX_LEN, EMBED_DIM),
                      (IDX_PER_WORKER // MAX_LEN, 1))
    out = _sc_embed(idx, embed_weight, pe_rep)
    return out.reshape(BATCH, MAX_LEN, EMBED_DIM)


# speedup vs baseline: 6.5500x; 1.5037x over previous
import functools

import jax
import jax.numpy as jnp
from jax import lax
from jax.experimental import pallas as pl
from jax.experimental.pallas import tpu as pltpu
from jax.experimental.pallas import tpu_sc as plsc

MAX_LEN = 200
EMBED_DIM = 128
BATCH = 1024
NUM_CORES = 2
NUM_SUBCORES = 16
NUM_WORKERS = NUM_CORES * NUM_SUBCORES
CHUNK = 128
IDX_PER_WORKER = BATCH * MAX_LEN // NUM_WORKERS
NUM_CHUNKS = IDX_PER_WORKER // CHUNK
UNROLL = 2
LANES = 16


def _sc_embed(idx, table, pe2d):
    mesh = plsc.VectorSubcoreMesh(core_axis_name="c", subcore_axis_name="s")

    @functools.partial(
        pl.kernel,
        mesh=mesh,
        out_type=jax.ShapeDtypeStruct((BATCH * MAX_LEN, EMBED_DIM), jnp.float32),
        scratch_types=[
            pltpu.VMEM((NUM_CHUNKS, CHUNK), jnp.int32),
            pltpu.VMEM((CHUNK, EMBED_DIM), jnp.float32),
            pltpu.VMEM((CHUNK, EMBED_DIM), jnp.float32),
            pltpu.VMEM_SHARED((IDX_PER_WORKER, EMBED_DIM), jnp.float32),
            pltpu.SemaphoreType.DMA,
            pltpu.SemaphoreType.DMA,
        ],
    )
    def k(idx_hbm, table_hbm, pe_hbm, out_hbm, idx_v, buf0, buf1, pe_sh,
          sem0, sem1):
        sid = lax.axis_index("s")
        wid = sid * NUM_CORES + lax.axis_index("c")
        base = wid * IDX_PER_WORKER
        pltpu.sync_copy(idx_hbm.at[wid], idx_v)

        PE_SLICE = IDX_PER_WORKER // NUM_SUBCORES
        pltpu.sync_copy(pe_hbm.at[pl.ds(sid * PE_SLICE, PE_SLICE)],
                        pe_sh.at[pl.ds(sid * PE_SLICE, PE_SLICE)])
        plsc.subcore_barrier()

        bufs = (buf0, buf1)
        sems = (sem0, sem1)

        pltpu.sync_copy(pe_sh.at[pl.ds(0, CHUNK)], buf0)
        pltpu.async_copy(table_hbm.at[idx_v.at[0]], buf0, sem0, add=True)

        @pl.loop(0, NUM_CHUNKS, step=UNROLL)
        def _(g):
            for b in range(UNROLL):
                gg = g + b
                buf, sem = bufs[b % 2], sems[b % 2]
                nbuf, nsem = bufs[1 - b % 2], sems[1 - b % 2]

                @pl.when(gg + 1 < NUM_CHUNKS)
                def _():
                    pltpu.sync_copy(pe_sh.at[pl.ds((gg + 1) * CHUNK, CHUNK)],
                                    nbuf)
                    pltpu.async_copy(table_hbm.at[idx_v.at[gg + 1]], nbuf, nsem,
                                     add=True)

                pltpu.make_async_copy(table_hbm.at[idx_v.at[gg]], buf, sem).wait()
                pltpu.sync_copy(buf, out_hbm.at[pl.ds(base + gg * CHUNK, CHUNK)])

    return k(idx, table, pe2d)


def kernel(x, embed_weight, pe):
    idx = x.astype(jnp.int32).reshape(NUM_WORKERS, NUM_CHUNKS, CHUNK)
    pe_rep = jnp.tile(pe.reshape(MAX_LEN, EMBED_DIM),
                      (IDX_PER_WORKER // MAX_LEN, 1))
    out = _sc_embed(idx, embed_weight, pe_rep)
    return out.reshape(BATCH, MAX_LEN, EMBED_DIM)


# speedup vs baseline: 7.2151x; 1.1015x over previous
import functools

import jax
import jax.numpy as jnp
from jax import lax
from jax.experimental import pallas as pl
from jax.experimental.pallas import tpu as pltpu
from jax.experimental.pallas import tpu_sc as plsc

MAX_LEN = 200
EMBED_DIM = 128
BATCH = 1024
NUM_CORES = 2
NUM_SUBCORES = 16
NUM_WORKERS = NUM_CORES * NUM_SUBCORES
CHUNK = 128
IDX_PER_WORKER = BATCH * MAX_LEN // NUM_WORKERS
NUM_CHUNKS = IDX_PER_WORKER // CHUNK
NBUF = 4
LANES = 16


def _sc_embed(idx, table, pe2d):
    mesh = plsc.VectorSubcoreMesh(core_axis_name="c", subcore_axis_name="s")

    @functools.partial(
        pl.kernel,
        mesh=mesh,
        out_type=jax.ShapeDtypeStruct((BATCH * MAX_LEN, EMBED_DIM), jnp.float32),
        scratch_types=[
            pltpu.VMEM((NUM_CHUNKS, CHUNK), jnp.int32),
            pltpu.VMEM((CHUNK, EMBED_DIM), jnp.float32),
            pltpu.VMEM((CHUNK, EMBED_DIM), jnp.float32),
            pltpu.VMEM((CHUNK, EMBED_DIM), jnp.float32),
            pltpu.VMEM((CHUNK, EMBED_DIM), jnp.float32),
            pltpu.VMEM_SHARED((IDX_PER_WORKER, EMBED_DIM), jnp.float32),
            pltpu.SemaphoreType.DMA,
            pltpu.SemaphoreType.DMA,
            pltpu.SemaphoreType.DMA,
            pltpu.SemaphoreType.DMA,
            pltpu.SemaphoreType.DMA,
            pltpu.SemaphoreType.DMA,
            pltpu.SemaphoreType.DMA,
            pltpu.SemaphoreType.DMA,
        ],
    )
    def k(idx_hbm, table_hbm, pe_hbm, out_hbm, idx_v, buf0, buf1, buf2, buf3,
          pe_sh, gsem0, gsem1, gsem2, gsem3, fsem0, fsem1, fsem2, fsem3):
        sid = lax.axis_index("s")
        wid = sid * NUM_CORES + lax.axis_index("c")
        base = wid * IDX_PER_WORKER
        pltpu.sync_copy(idx_hbm.at[wid], idx_v)

        PE_SLICE = IDX_PER_WORKER // NUM_SUBCORES
        pltpu.sync_copy(pe_hbm.at[pl.ds(sid * PE_SLICE, PE_SLICE)],
                        pe_sh.at[pl.ds(sid * PE_SLICE, PE_SLICE)])
        plsc.subcore_barrier()

        bufs = (buf0, buf1, buf2, buf3)
        gsems = (gsem0, gsem1, gsem2, gsem3)
        fsems = (fsem0, fsem1, fsem2, fsem3)

        def prep(gg):
            s = gg % NBUF
            pltpu.sync_copy(pe_sh.at[pl.ds(gg * CHUNK, CHUNK)], bufs[s])
            pltpu.async_copy(table_hbm.at[idx_v.at[gg]], bufs[s], gsems[s],
                             add=True)

        def finish(gg):
            s = gg % NBUF
            pltpu.make_async_copy(table_hbm.at[idx_v.at[gg]], bufs[s],
                                  gsems[s]).wait()
            pltpu.async_copy(bufs[s], out_hbm.at[pl.ds(base + gg * CHUNK, CHUNK)],
                             fsems[s])

        def wait_flush(gg):
            s = gg % NBUF
            pltpu.make_async_copy(bufs[s],
                                  out_hbm.at[pl.ds(base + gg * CHUNK, CHUNK)],
                                  fsems[s]).wait()

        prep(0)
        prep(1)
        for gg in range(NUM_CHUNKS):
            nxt = gg + 2
            if nxt < NUM_CHUNKS:
                if nxt - NBUF >= 0:
                    wait_flush(nxt - NBUF)
                prep(nxt)
            finish(gg)
        for gg in range(max(0, NUM_CHUNKS - NBUF), NUM_CHUNKS):
            wait_flush(gg)

    return k(idx, table, pe2d)


def kernel(x, embed_weight, pe):
    idx = x.astype(jnp.int32).reshape(NUM_WORKERS, NUM_CHUNKS, CHUNK)
    pe_rep = jnp.tile(pe.reshape(MAX_LEN, EMBED_DIM),
                      (IDX_PER_WORKER // MAX_LEN, 1))
    out = _sc_embed(idx, embed_weight, pe_rep)
    return out.reshape(BATCH, MAX_LEN, EMBED_DIM)


# speedup vs baseline: 7.5168x; 1.0418x over previous
import functools

import jax
import jax.numpy as jnp
from jax import lax
from jax.experimental import pallas as pl
from jax.experimental.pallas import tpu as pltpu
from jax.experimental.pallas import tpu_sc as plsc

MAX_LEN = 200
EMBED_DIM = 128
BATCH = 1024
NUM_CORES = 2
NUM_SUBCORES = 16
NUM_WORKERS = NUM_CORES * NUM_SUBCORES
CHUNK = 128
IDX_PER_WORKER = BATCH * MAX_LEN // NUM_WORKERS
NUM_CHUNKS = IDX_PER_WORKER // CHUNK
NBUF = 6
DEPTH = 3
LANES = 16


def _sc_embed(idx, table, pe2d):
    mesh = plsc.VectorSubcoreMesh(core_axis_name="c", subcore_axis_name="s")

    @functools.partial(
        pl.kernel,
        mesh=mesh,
        out_type=jax.ShapeDtypeStruct((BATCH * MAX_LEN, EMBED_DIM), jnp.float32),
        scratch_types=[
            pltpu.VMEM((NUM_CHUNKS, CHUNK), jnp.int32),
            *([pltpu.VMEM((CHUNK, EMBED_DIM), jnp.float32)] * NBUF),
            pltpu.VMEM_SHARED((MAX_LEN, EMBED_DIM), jnp.float32),
            *([pltpu.SemaphoreType.DMA] * (2 * NBUF)),
        ],
    )
    def k(idx_hbm, table_hbm, pe_hbm, out_hbm, idx_v, *rest):
        bufs = rest[:NBUF]
        pe_sh = rest[NBUF]
        gsems = rest[NBUF + 1:2 * NBUF + 1]
        fsems = rest[2 * NBUF + 1:3 * NBUF + 1]
        sid = lax.axis_index("s")
        wid = sid * NUM_CORES + lax.axis_index("c")
        base = wid * IDX_PER_WORKER
        pltpu.sync_copy(idx_hbm.at[wid], idx_v)

        @pl.when(sid == 0)
        def _():
            pltpu.sync_copy(pe_hbm, pe_sh)

        plsc.subcore_barrier()

        def prep(gg):
            s = gg % NBUF
            p0 = (gg * CHUNK) % MAX_LEN
            n1 = min(MAX_LEN - p0, CHUNK)
            pltpu.sync_copy(pe_sh.at[pl.ds(p0, n1)], bufs[s].at[pl.ds(0, n1)])
            if n1 < CHUNK:
                pltpu.sync_copy(pe_sh.at[pl.ds(0, CHUNK - n1)],
                                bufs[s].at[pl.ds(n1, CHUNK - n1)])
            pltpu.async_copy(table_hbm.at[idx_v.at[gg]], bufs[s], gsems[s],
                             add=True)

        def finish(gg):
            s = gg % NBUF
            pltpu.make_async_copy(table_hbm.at[idx_v.at[gg]], bufs[s],
                                  gsems[s]).wait()
            pltpu.async_copy(bufs[s], out_hbm.at[pl.ds(base + gg * CHUNK, CHUNK)],
                             fsems[s])

        def wait_flush(gg):
            s = gg % NBUF
            pltpu.make_async_copy(bufs[s],
                                  out_hbm.at[pl.ds(base + gg * CHUNK, CHUNK)],
                                  fsems[s]).wait()

        for gg in range(DEPTH):
            prep(gg)
        for gg in range(NUM_CHUNKS):
            nxt = gg + DEPTH
            if nxt < NUM_CHUNKS:
                if nxt - NBUF >= 0:
                    wait_flush(nxt - NBUF)
                prep(nxt)
            finish(gg)
        for gg in range(max(0, NUM_CHUNKS - NBUF), NUM_CHUNKS):
            wait_flush(gg)

    return k(idx, table, pe2d)


def kernel(x, embed_weight, pe):
    idx = x.astype(jnp.int32).reshape(NUM_WORKERS, NUM_CHUNKS, CHUNK)
    out = _sc_embed(idx, embed_weight, pe.reshape(MAX_LEN, EMBED_DIM))
    return out.reshape(BATCH, MAX_LEN, EMBED_DIM)
